# baseline (speedup 1.0000x reference)
import functools

import jax
import jax.numpy as jnp
from jax import lax
from jax.experimental import pallas as pl
from jax.experimental.pallas import tpu as pltpu
from jax.experimental.pallas import tpu_sc as plsc

NC = 2
NS = 16
NW = NC * NS
GW = 16
D = 32
SL = 8


def _make_sc_kernel(B, b_per_w, n_groups):
    mesh = plsc.VectorSubcoreMesh(core_axis_name="c", subcore_axis_name="s")

    @functools.partial(
        pl.kernel,
        mesh=mesh,
        compiler_params=pltpu.CompilerParams(
            use_tc_tiling_on_sc=True, needs_layout_passes=False),
        out_type=jax.ShapeDtypeStruct((B,), jnp.float32),
        scratch_types=[
            pltpu.VMEM((b_per_w // 128, 128), jnp.int32),
            pltpu.VMEM((b_per_w // 128, 128), jnp.int32),
            pltpu.VMEM((2, 4, SL, 128), jnp.float32),
            pltpu.VMEM((2, 4, SL, 128), jnp.float32),
            pltpu.VMEM((2, 4, SL, 128), jnp.float32),
            pltpu.VMEM((2, 4, SL, 128), jnp.float32),
            pltpu.VMEM((2, 4, SL, 128), jnp.float32),
            pltpu.VMEM((2, 4, SL, 128), jnp.float32),
            pltpu.VMEM((2, 4, SL, 128), jnp.float32),
            pltpu.VMEM((2, 4, SL, 128), jnp.float32),
            pltpu.VMEM((256,), jnp.float32),
            pltpu.VMEM((256,), jnp.float32),
            pltpu.VMEM((256,), jnp.float32),
            pltpu.VMEM((256,), jnp.float32),
            pltpu.VMEM((b_per_w,), jnp.float32),
            pltpu.SemaphoreType.DMA,
            pltpu.SemaphoreType.DMA,
            pltpu.SemaphoreType.DMA,
            pltpu.SemaphoreType.DMA,
        ],
    )
    def sc_kernel(stu_hbm, exer_hbm, th3_hbm, a3_hbm, b1_hbm, out_hbm,
                  sidx, eidx, th0, th1, th2, th3b, a0, a1, a2, a3b,
                  b0, b1, b2, b3, res, sem0, sem1, sem2, sem3):
        ths = (th0, th1, th2, th3b)
        aas = (a0, a1, a2, a3b)
        bbs = (b0, b1, b2, b3)
        sems = (sem0, sem1, sem2, sem3)
        wid = lax.axis_index("s") * NC + lax.axis_index("c")
        base = wid * b_per_w
        n_irows = b_per_w // 128
        grow = wid * n_irows
        pltpu.sync_copy(stu_hbm.at[pl.ds(grow, n_irows)], sidx)
        pltpu.sync_copy(exer_hbm.at[pl.ds(grow, n_irows)], eidx)

        lanes = lax.broadcasted_iota(jnp.int32, (GW,), 0)

        def load_idx(ref, g):
            return ref[g // 8, pl.ds((g % 8) * GW, GW)]

        def issue_group(g, th_buf, a_buf, b_buf, sem):
            vs = load_idx(sidx, g)
            ve = load_idx(eidx, g)

            @pl.loop(0, GW)
            def _(j):
                m = lanes == j
                sj = jnp.sum(jnp.where(m, vs, 0))
                ej = jnp.sum(jnp.where(m, ve, 0))
                s_start = (sj // GW) * GW
                e_start = (ej // GW) * GW
                half, slot = j // 8, (j % 8) * GW
                pltpu.async_copy(
                    th3_hbm.at[:, :, pl.ds(s_start, GW)],
                    th_buf.at[half, :, :, pl.ds(slot, GW)], sem)
                pltpu.async_copy(
                    a3_hbm.at[:, :, pl.ds(e_start, GW)],
                    a_buf.at[half, :, :, pl.ds(slot, GW)], sem)
                pltpu.async_copy(
                    b1_hbm.at[pl.ds(e_start, GW)],
                    b_buf.at[pl.ds(j * GW, GW)], sem)

        def drain_group(th_buf, a_buf, b_buf, sem):
            dummy3 = th3_hbm.at[:, :, pl.ds(0, 128)]
            for buf in (th_buf, a_buf):
                pltpu.make_async_copy(dummy3, buf.at[0], sem).wait()
                pltpu.make_async_copy(dummy3, buf.at[1], sem).wait()
            pltpu.make_async_copy(b1_hbm.at[pl.ds(0, 256)], b_buf, sem).wait()

        def compute_group(g, th_buf, a_buf, b_buf):
            vs = load_idx(sidx, g)
            ve = load_idx(eidx, g)
            half_v = lanes // 8
            s_lane = (lanes % 8) * GW + lax.rem(vs, GW)
            e_lane = (lanes % 8) * GW + lax.rem(ve, GW)
            zero_v = jnp.zeros((GW,), jnp.int32)

            def dbody(d, acc):
                d0 = zero_v + d // SL
                d1 = zero_v + lax.rem(d, SL)
                th_v = plsc.load_gather(th_buf, [half_v, d0, d1, s_lane])
                a_v = plsc.load_gather(a_buf, [half_v, d0, d1, e_lane])
                asig = 1.0 / (1.0 + jnp.exp(-a_v))
                return acc + asig * th_v

            acc = lax.fori_loop(0, D, dbody, jnp.zeros((GW,), jnp.float32))
            b_v = plsc.load_gather(b_buf, [lanes * GW + lax.rem(ve, GW)])
            logit = acc - b_v
            res[pl.ds(g * GW, GW)] = 1.0 / (1.0 + jnp.exp(-logit))

        NB = 4
        for k in range(NB - 1):
            issue_group(k, ths[k], aas[k], bbs[k], sems[k])

        @pl.loop(0, (n_groups - NB) // NB)
        def _(i):
            gb = i * NB
            for k in range(NB):
                g = gb + k
                ka = (k + NB - 1) % NB
                issue_group(g + NB - 1, ths[ka], aas[ka], bbs[ka], sems[ka])
                drain_group(ths[k], aas[k], bbs[k], sems[k])
                compute_group(g, ths[k], aas[k], bbs[k])

        for g in range(n_groups - NB, n_groups):
            k = g % NB
            ka = (k + NB - 1) % NB
            if g + NB - 1 < n_groups:
                issue_group(g + NB - 1, ths[ka], aas[ka], bbs[ka], sems[ka])
            drain_group(ths[k], aas[k], bbs[k], sems[k])
            compute_group(g, ths[k], aas[k], bbs[k])

        pltpu.sync_copy(res, out_hbm.at[pl.ds(base, b_per_w)])

    return sc_kernel


def kernel(stu_id, input_exercise, theta_w, a_w, b_w):
    B = stu_id.shape[0]
    b_per_w = B // NW
    n_groups = b_per_w // GW
    stu2 = stu_id.astype(jnp.int32).reshape(B // 128, 128)
    exer2 = input_exercise.astype(jnp.int32).reshape(B // 128, 128)

    th3 = jnp.transpose(theta_w).reshape(4, SL, theta_w.shape[0])
    a3 = jnp.transpose(a_w).reshape(4, SL, a_w.shape[0])
    b1 = b_w.reshape(b_w.shape[0])

    sc_kernel = _make_sc_kernel(B, b_per_w, n_groups)
    return sc_kernel(stu2, exer2, th3, a3, b1)
